# baseline (device time: 51629 ns/iter reference)
import jax
import jax.numpy as jnp
from jax import lax
from jax.experimental import pallas as pl
from jax.experimental.pallas import tpu as pltpu

N_DEV = 8
B_PER = 2
SQ = 256
D_MODEL = 512
H_PER = 4
DH = 64
HG = H_PER * DH
BLK = 64

_R_SRC, _R_DST = (0, 1, 2), (1, 2, 3)
_L_SRC, _L_DST = (0, 7), (7, 6)
_Z_SRC, _Z_DST = (0, 1), (4, 5)


def _body(x_ref, w_ref, k_ref, v_ref, out_ref, wg_ref,
          r_send, r_recv, l_send, l_recv, z_send, z_recv):
    my = lax.axis_index("i")
    left = lax.rem(my + N_DEV - 1, N_DEV)
    right = lax.rem(my + 1, N_DEV)
    zpeer = lax.rem(my + 4, N_DEV)

    barrier_sem = pltpu.get_barrier_semaphore()
    for nbr in (left, right, zpeer):
        pl.semaphore_signal(
            barrier_sem, inc=1,
            device_id=(nbr,), device_id_type=pl.DeviceIdType.MESH,
        )
    pl.semaphore_wait(barrier_sem, 3)

    wg_ref[0] = w_ref[...]

    x2b = x_ref[...].reshape(B_PER * SQ, D_MODEL).astype(jnp.bfloat16)

    qblk = lax.broadcasted_iota(jnp.int32, (SQ, SQ), 0) // BLK
    kblk = lax.broadcasted_iota(jnp.int32, (SQ, SQ), 1) // BLK
    mask = kblk <= qblk

    def compute_chunk(r, first=False):
        wq = wg_ref[r, 0:D_MODEL, :]
        wot = wg_ref[r, D_MODEL:, :]
        q2 = jnp.dot(x2b, wq, preferred_element_type=jnp.float32)
        ctx_rows = []
        for b in range(B_PER):
            heads = []
            for h in range(H_PER):
                q = q2[b * SQ:(b + 1) * SQ, h * DH:(h + 1) * DH]
                k = k_ref[r, b, :, h * DH:(h + 1) * DH]
                v = v_ref[r, b, :, h * DH:(h + 1) * DH]
                s = lax.dot_general(
                    q, k, (((1,), (1,)), ((), ())),
                    preferred_element_type=jnp.float32,
                )
                e = jnp.where(mask, jnp.exp(s), 0.0)
                recip = 1.0 / jnp.sum(e, axis=1, keepdims=True)
                ctx = jnp.dot(e, v,
                              preferred_element_type=jnp.float32) * recip
                heads.append(ctx)
            ctx_rows.append(jnp.concatenate(heads, axis=1))
        ctx2 = jnp.concatenate(ctx_rows, axis=0).astype(jnp.bfloat16)
        contrib = lax.dot_general(
            ctx2, wot, (((1,), (1,)), ((), ())),
            preferred_element_type=jnp.float32,
        ).reshape(B_PER, SQ, D_MODEL)
        if first:
            out_ref[...] = contrib
        else:
            out_ref[...] = out_ref[...] + contrib

    def stream_copy(src_slot, dst_slot, send_sem, recv_sem, peer):
        return pltpu.make_async_remote_copy(
            src_ref=wg_ref.at[src_slot], dst_ref=wg_ref.at[dst_slot],
            send_sem=send_sem, recv_sem=recv_sem,
            device_id=(peer,), device_id_type=pl.DeviceIdType.MESH,
        )

    for t in range(3):
        rr = stream_copy(_R_SRC[t], _R_DST[t], r_send.at[t], r_recv.at[t],
                         right)
        rr.start()
        rl = rz = None
        if t < 2:
            rl = stream_copy(_L_SRC[t], _L_DST[t], l_send.at[t],
                             l_recv.at[t], left)
            rz = stream_copy(_Z_SRC[t], _Z_DST[t], z_send.at[t],
                             z_recv.at[t], zpeer)
            rl.start()
            rz.start()
        if t == 0:
            compute_chunk(0, first=True)
        elif t == 1:
            compute_chunk(1)
            compute_chunk(7)
            compute_chunk(4)
        else:
            compute_chunk(2)
            compute_chunk(6)
            compute_chunk(5)
        rr.wait()
        if rl is not None:
            rl.wait()
        if rz is not None:
            rz.wait()

    compute_chunk(3)


def kernel(x, Wq, K_ext, V_ext, Wo):
    my = lax.axis_index("i")

    wpack = jnp.concatenate([Wq * 0.125, Wo.T], axis=0).astype(jnp.bfloat16)

    kb = lax.dynamic_slice_in_dim(K_ext, B_PER * my, B_PER, axis=0)
    vb = lax.dynamic_slice_in_dim(V_ext, B_PER * my, B_PER, axis=0)

    idx = jnp.mod(my - jnp.arange(N_DEV), N_DEV)
    kr = jnp.moveaxis(
        jnp.take(kb.reshape(B_PER, SQ, N_DEV, HG), idx, axis=2), 2, 0)
    vr = jnp.moveaxis(
        jnp.take(vb.reshape(B_PER, SQ, N_DEV, HG), idx, axis=2), 2, 0)

    return pl.pallas_call(
        _body,
        out_shape=jax.ShapeDtypeStruct((B_PER, SQ, D_MODEL), jnp.float32),
        in_specs=[pl.BlockSpec(memory_space=pltpu.VMEM)] * 4,
        out_specs=pl.BlockSpec(memory_space=pltpu.VMEM),
        scratch_shapes=[
            pltpu.VMEM((N_DEV, 2 * D_MODEL, HG), jnp.bfloat16),
            pltpu.SemaphoreType.DMA((3,)),
            pltpu.SemaphoreType.DMA((3,)),
            pltpu.SemaphoreType.DMA((2,)),
            pltpu.SemaphoreType.DMA((2,)),
            pltpu.SemaphoreType.DMA((2,)),
            pltpu.SemaphoreType.DMA((2,)),
        ],
        compiler_params=pltpu.CompilerParams(collective_id=0),
    )(x, wpack, kr, vr)
